# Initial kernel scaffold; baseline (speedup 1.0000x reference)
import jax
import jax.numpy as jnp
from jax.experimental import pallas as pl


def kernel(h_node, x, ptr, W1, b1, W2, b2):
    raise NotImplementedError("write your pallas kernel here")


# speedup vs baseline: 3.8327x; 3.8327x over previous
import functools

import jax
import jax.numpy as jnp
from jax.experimental import pallas as pl
from jax.experimental.pallas import tpu as pltpu


def _fused_body(lo_ref, hi_ref, w1x_ref, w1h_ref, w2_ref, b1_ref, b2_ref,
                x_ref, h_ref, out_ref, *, block_rows):
    pid = pl.program_id(0)
    hidden = jnp.maximum(
        jnp.dot(x_ref[...], w1x_ref[...], preferred_element_type=jnp.float32)
        + jnp.dot(h_ref[...], w1h_ref[...], preferred_element_type=jnp.float32)
        + b1_ref[...], 0.0)
    rows = jnp.dot(hidden, w2_ref[...],
                   preferred_element_type=jnp.float32) + b2_ref[...]
    r0 = pid * block_rows
    ridx = r0 + jax.lax.broadcasted_iota(jnp.int32, (1, block_rows), 1)
    mask = jnp.logical_and(ridx >= lo_ref[...], ridx < hi_ref[...])
    partial = jnp.dot(mask.astype(jnp.float32), rows,
                      preferred_element_type=jnp.float32)

    @pl.when(pid == 0)
    def _init():
        out_ref[...] = jnp.zeros_like(out_ref)

    out_ref[...] += partial


def kernel(h_node, x, ptr, W1, b1, W2, b2):
    n, embed_dim = h_node.shape
    nfeat = x.shape[1]
    nseg = ptr.shape[0] - 1
    hidden_dim = W1.shape[1]

    block_rows = 512
    assert n % block_rows == 0
    grid = (n // block_rows,)

    ptr32 = ptr.astype(jnp.int32)
    lo = ptr32[:-1].reshape(nseg, 1)
    hi = ptr32[1:].reshape(nseg, 1)
    w1x = W1[:nfeat]
    w1h = W1[nfeat:]

    out = pl.pallas_call(
        functools.partial(_fused_body, block_rows=block_rows),
        grid=grid,
        in_specs=[
            pl.BlockSpec((nseg, 1), lambda i: (0, 0)),
            pl.BlockSpec((nseg, 1), lambda i: (0, 0)),
            pl.BlockSpec((nfeat, hidden_dim), lambda i: (0, 0)),
            pl.BlockSpec((embed_dim, hidden_dim), lambda i: (0, 0)),
            pl.BlockSpec((hidden_dim, embed_dim), lambda i: (0, 0)),
            pl.BlockSpec((1, hidden_dim), lambda i: (0, 0)),
            pl.BlockSpec((1, embed_dim), lambda i: (0, 0)),
            pl.BlockSpec((block_rows, nfeat), lambda i: (i, 0)),
            pl.BlockSpec((block_rows, embed_dim), lambda i: (i, 0)),
        ],
        out_specs=pl.BlockSpec((nseg, embed_dim), lambda i: (0, 0)),
        out_shape=jax.ShapeDtypeStruct((nseg, embed_dim), jnp.float32),
        compiler_params=pltpu.CompilerParams(
            dimension_semantics=("arbitrary",),
        ),
    )(lo, hi, w1x, w1h, W2, b1.reshape(1, -1), b2.reshape(1, -1), x, h_node)
    return out
